# baseline (speedup 1.0000x reference)
import functools

import jax
import jax.numpy as jnp
from jax import lax
from jax.experimental import pallas as pl
from jax.experimental.pallas import tpu as pltpu

DIM = 64
B = 16384
CH = 4096
NCH = B // CH


def _gather_kernel(user_smem, item_smem, ut_hbm, it_hbm, out_u, out_i,
                   buf_a, buf_b, sem_g0, sem_g1, sem_g2, sem_g3,
                   sem_wa, sem_wb):
    gsems = (sem_g0, sem_g1, sem_g2, sem_g3)
    bufs = (buf_a, buf_b)
    wsems = (sem_wa, sem_wb)
    plan = [(user_smem, ut_hbm, out_u), (item_smem, it_hbm, out_i)]
    pending = {}

    g = 0
    for idx_smem, tbl, out in plan:
        for c in range(NCH):
            par = g % 2
            buf, wsem = bufs[par], wsems[par]
            if par in pending:
                pbuf, pout, pbase = pending.pop(par)
                pltpu.make_async_copy(
                    pbuf, pout.at[pl.ds(pbase, CH)], wsem
                ).wait()

            def fire(j4, _, idx_smem=idx_smem, tbl=tbl, buf=buf, c=c):
                for q in range(4):
                    j = j4 * 4 + q
                    i = idx_smem[c * CH + j]
                    pltpu.make_async_copy(
                        tbl.at[pl.ds(i, 1)], buf.at[pl.ds(j, 1)], gsems[q]
                    ).start()
                return ()

            lax.fori_loop(0, CH // 4, fire, (), unroll=4)

            for q in range(4):
                pltpu.make_async_copy(
                    tbl.at[pl.ds(0, CH // 4)],
                    buf.at[pl.ds(q * (CH // 4), CH // 4)],
                    gsems[q],
                ).wait()

            pltpu.make_async_copy(
                buf, out.at[pl.ds(c * CH, CH)], wsem
            ).start()
            pending[par] = (buf, out, c * CH)
            g += 1

    for par, (pbuf, pout, pbase) in pending.items():
        pltpu.make_async_copy(
            pbuf, pout.at[pl.ds(pbase, CH)], wsems[par]
        ).wait()


@functools.lru_cache(maxsize=None)
def _build_kernel():
    return pl.pallas_call(
        _gather_kernel,
        out_shape=(
            jax.ShapeDtypeStruct((B, DIM), jnp.float32),
            jax.ShapeDtypeStruct((B, DIM), jnp.float32),
        ),
        in_specs=[
            pl.BlockSpec(memory_space=pltpu.SMEM),
            pl.BlockSpec(memory_space=pltpu.SMEM),
            pl.BlockSpec(memory_space=pl.ANY),
            pl.BlockSpec(memory_space=pl.ANY),
        ],
        out_specs=(
            pl.BlockSpec(memory_space=pl.ANY),
            pl.BlockSpec(memory_space=pl.ANY),
        ),
        scratch_shapes=[
            pltpu.VMEM((CH, DIM), jnp.float32),
            pltpu.VMEM((CH, DIM), jnp.float32),
            pltpu.SemaphoreType.DMA,
            pltpu.SemaphoreType.DMA,
            pltpu.SemaphoreType.DMA,
            pltpu.SemaphoreType.DMA,
            pltpu.SemaphoreType.DMA,
            pltpu.SemaphoreType.DMA,
        ],
    )


def kernel(user, item, user_table, item_table):
    return _build_kernel()(user, item, user_table, item_table)


# speedup vs baseline: 1.1530x; 1.1530x over previous
import functools

import jax
import jax.numpy as jnp
from jax import lax
from jax.experimental import pallas as pl
from jax.experimental.pallas import tpu as pltpu
from jax.experimental.pallas import tpu_sc as plsc

DIM = 64
B = 16384
WAVE = 16


@functools.lru_cache(maxsize=None)
def _build_kernel():
    info = plsc.get_sparse_core_info()
    nc, ns = info.num_cores, info.num_subcores
    nw = nc * ns
    b_per_w = B // nw
    chunk = b_per_w // 2
    mesh = plsc.VectorSubcoreMesh(core_axis_name="c", subcore_axis_name="s")

    @functools.partial(
        pl.kernel,
        mesh=mesh,
        out_type=(
            jax.ShapeDtypeStruct((B, DIM), jnp.float32),
            jax.ShapeDtypeStruct((B, DIM), jnp.float32),
        ),
        scratch_types=[
            pltpu.VMEM((b_per_w,), jnp.int32),
            pltpu.VMEM((b_per_w,), jnp.int32),
            pltpu.VMEM((chunk, DIM), jnp.float32),
            pltpu.VMEM((chunk, DIM), jnp.float32),
            pltpu.SemaphoreType.DMA,
            pltpu.SemaphoreType.DMA,
        ],
    )
    def gather_kernel(user_hbm, item_hbm, ut_hbm, it_hbm, out_u, out_i,
                      idx_u, idx_i, rows_u, rows_i, sem_u, sem_i):
        wid = lax.axis_index("s") * nc + lax.axis_index("c")
        base = wid * b_per_w
        pltpu.sync_copy(user_hbm.at[pl.ds(base, b_per_w)], idx_u)
        pltpu.sync_copy(item_hbm.at[pl.ds(base, b_per_w)], idx_i)

        def fire_chunk(tbl_hbm, idx_v, rows_v, sem, c0):
            def wave(w, _):
                j0 = w * WAVE
                vec = idx_v[pl.ds(c0 + j0, WAVE)]
                for k in range(WAVE):
                    pltpu.async_copy(tbl_hbm.at[vec[k]], rows_v.at[j0 + k],
                                     sem)
                return ()

            lax.fori_loop(0, chunk // WAVE, wave, (), unroll=False)

        def drain_chunk(tbl_hbm, rows_v, sem):
            pltpu.make_async_copy(
                tbl_hbm.at[pl.ds(0, chunk)], rows_v, sem
            ).wait()

        def write_chunk(rows_v, out, c0):
            pltpu.sync_copy(rows_v, out.at[pl.ds(base + c0, chunk)])

        fire_chunk(ut_hbm, idx_u, rows_u, sem_u, 0)
        fire_chunk(it_hbm, idx_i, rows_i, sem_i, 0)
        drain_chunk(ut_hbm, rows_u, sem_u)
        write_chunk(rows_u, out_u, 0)
        fire_chunk(ut_hbm, idx_u, rows_u, sem_u, chunk)
        drain_chunk(it_hbm, rows_i, sem_i)
        write_chunk(rows_i, out_i, 0)
        fire_chunk(it_hbm, idx_i, rows_i, sem_i, chunk)
        drain_chunk(ut_hbm, rows_u, sem_u)
        write_chunk(rows_u, out_u, chunk)
        drain_chunk(it_hbm, rows_i, sem_i)
        write_chunk(rows_i, out_i, chunk)

    return gather_kernel


def kernel(user, item, user_table, item_table):
    return _build_kernel()(user, item, user_table, item_table)
